# Initial kernel scaffold; baseline (speedup 1.0000x reference)
import jax
import jax.numpy as jnp
from jax.experimental import pallas as pl


def kernel(x, embedding_matrix):
    raise NotImplementedError("write your pallas kernel here")


# speedup vs baseline: 4.9474x; 4.9474x over previous
import functools

import jax
import jax.numpy as jnp
from jax import lax
from jax.experimental import pallas as pl
from jax.experimental.pallas import tpu as pltpu
from jax.experimental.pallas import tpu_sc as plsc


def _make_gather(n_tok: int, emb_dim: int):
    info = plsc.get_sparse_core_info()
    nw = info.num_cores * info.num_subcores
    per_w = n_tok // nw
    chunk = 2048
    while per_w % chunk:
        chunk //= 2
    n_chunks = per_w // chunk

    mesh = plsc.VectorSubcoreMesh(core_axis_name="c", subcore_axis_name="s")

    @functools.partial(
        pl.kernel,
        mesh=mesh,
        out_type=jax.ShapeDtypeStruct((n_tok, emb_dim), jnp.float32),
        scratch_types=[
            pltpu.VMEM((chunk,), jnp.int32),
            pltpu.VMEM((chunk, emb_dim), jnp.float32),
            pltpu.SemaphoreType.DMA,
        ],
        compiler_params=pltpu.CompilerParams(use_tc_tiling_on_sc=False),
    )
    def gather(table_hbm, idx_hbm, out_hbm, idx_v, rows_v, sem):
        wid = lax.axis_index("s") * info.num_cores + lax.axis_index("c")
        base = wid * per_w

        def body(i, carry):
            off = base + i * chunk
            pltpu.sync_copy(idx_hbm.at[pl.ds(off, chunk)], idx_v)
            pltpu.async_copy(table_hbm.at[idx_v], rows_v, sem).wait()
            pltpu.sync_copy(rows_v, out_hbm.at[pl.ds(off, chunk)])
            return carry

        lax.fori_loop(0, n_chunks, body, 0)

    return gather


def kernel(x, embedding_matrix):
    b, l = x.shape
    n_emb, emb_dim = embedding_matrix.shape
    flat = x.reshape(-1).astype(jnp.int32)
    out = _make_gather(b * l, emb_dim)(embedding_matrix, flat)
    return out.reshape(b, l, emb_dim)


# speedup vs baseline: 5.0368x; 1.0181x over previous
import functools

import jax
import jax.numpy as jnp
from jax import lax
from jax.experimental import pallas as pl
from jax.experimental.pallas import tpu as pltpu
from jax.experimental.pallas import tpu_sc as plsc

_NBUF = 2


def _make_gather(n_tok: int, emb_dim: int):
    info = plsc.get_sparse_core_info()
    nw = info.num_cores * info.num_subcores
    per_w = n_tok // nw
    chunk = 1600
    while per_w % chunk:
        chunk //= 2
    n_chunks = per_w // chunk

    mesh = plsc.VectorSubcoreMesh(core_axis_name="c", subcore_axis_name="s")

    @functools.partial(
        pl.kernel,
        mesh=mesh,
        out_type=jax.ShapeDtypeStruct((n_tok, emb_dim), jnp.float32),
        scratch_types=[
            pltpu.VMEM((_NBUF, chunk), jnp.int32),
            pltpu.VMEM((_NBUF, chunk, emb_dim), jnp.float32),
        ]
        + [pltpu.SemaphoreType.DMA] * (2 * _NBUF + 1),
        compiler_params=pltpu.CompilerParams(use_tc_tiling_on_sc=False),
    )
    def gather(table_hbm, idx_hbm, out_hbm, idx_v, rows_v, *sems):
        isem = sems[:_NBUF]
        osem = sems[_NBUF : 2 * _NBUF]
        gsem = sems[2 * _NBUF]
        wid = lax.axis_index("s") * info.num_cores + lax.axis_index("c")
        base = wid * per_w

        def idx_copy(g, b):
            return pltpu.make_async_copy(
                idx_hbm.at[pl.ds(base + g * chunk, chunk)], idx_v.at[b], isem[b]
            )

        def out_copy(g, b):
            return pltpu.make_async_copy(
                rows_v.at[b], out_hbm.at[pl.ds(base + g * chunk, chunk)], osem[b]
            )

        for b in range(_NBUF):
            idx_copy(b, b).start()

        def body(g2, carry):
            for b in range(_NBUF):
                g = g2 * _NBUF + b
                idx_copy(g, b).wait()

                @pl.when(g2 > 0)
                def _():
                    out_copy(g, b).wait()

                pltpu.async_copy(table_hbm.at[idx_v.at[b]], rows_v.at[b], gsem).wait()

                @pl.when(g + _NBUF < n_chunks)
                def _():
                    idx_copy(g + _NBUF, b).start()

                out_copy(g, b).start()
            return carry

        lax.fori_loop(0, n_chunks // _NBUF, body, 0)
        for b in range(_NBUF):
            out_copy(0, b).wait()

    return gather


def kernel(x, embedding_matrix):
    b, l = x.shape
    n_emb, emb_dim = embedding_matrix.shape
    flat = x.reshape(-1).astype(jnp.int32)
    out = _make_gather(b * l, emb_dim)(embedding_matrix, flat)
    return out.reshape(b, l, emb_dim)


# speedup vs baseline: 5.0382x; 1.0003x over previous
import functools

import jax
import jax.numpy as jnp
from jax import lax
from jax.experimental import pallas as pl
from jax.experimental.pallas import tpu as pltpu
from jax.experimental.pallas import tpu_sc as plsc

_NBUF = 2
_ROWS = 8


def _make_gather(n_rows: int, seq: int, emb_dim: int):
    info = plsc.get_sparse_core_info()
    nw = info.num_cores * info.num_subcores
    rows_per_w = n_rows // nw
    n_chunks = rows_per_w // _ROWS

    mesh = plsc.VectorSubcoreMesh(core_axis_name="c", subcore_axis_name="s")

    @functools.partial(
        pl.kernel,
        mesh=mesh,
        out_type=jax.ShapeDtypeStruct((n_rows, seq, emb_dim), jnp.float32),
        scratch_types=[
            pltpu.VMEM((_NBUF, _ROWS, seq), jnp.int32),
            pltpu.VMEM((_NBUF, _ROWS, seq, emb_dim), jnp.float32),
        ]
        + [pltpu.SemaphoreType.DMA] * (2 * _NBUF + 1),
        compiler_params=pltpu.CompilerParams(use_tc_tiling_on_sc=False),
    )
    def gather(table_hbm, idx_hbm, out_hbm, idx_v, rows_v, *sems):
        isem = sems[:_NBUF]
        osem = sems[_NBUF : 2 * _NBUF]
        gsem = sems[2 * _NBUF]
        wid = lax.axis_index("s") * info.num_cores + lax.axis_index("c")
        base = wid * rows_per_w

        def idx_copy(g, b):
            return pltpu.make_async_copy(
                idx_hbm.at[pl.ds(base + g * _ROWS, _ROWS)], idx_v.at[b], isem[b]
            )

        def out_copy(g, b):
            return pltpu.make_async_copy(
                rows_v.at[b], out_hbm.at[pl.ds(base + g * _ROWS, _ROWS)], osem[b]
            )

        for b in range(_NBUF):
            idx_copy(b, b).start()

        def body(g2, carry):
            for b in range(_NBUF):
                g = g2 * _NBUF + b
                idx_copy(g, b).wait()

                @pl.when(g2 > 0)
                def _():
                    out_copy(g, b).wait()

                for r in range(_ROWS):
                    pltpu.make_async_copy(
                        table_hbm.at[idx_v.at[b].at[r]], rows_v.at[b].at[r], gsem
                    ).start()
                for r in range(_ROWS):
                    pltpu.make_async_copy(
                        table_hbm.at[idx_v.at[b].at[r]], rows_v.at[b].at[r], gsem
                    ).wait()

                @pl.when(g + _NBUF < n_chunks)
                def _():
                    idx_copy(g + _NBUF, b).start()

                out_copy(g, b).start()
            return carry

        lax.fori_loop(0, n_chunks // _NBUF, body, 0)
        for b in range(_NBUF):
            out_copy(0, b).wait()

    return gather


def kernel(x, embedding_matrix):
    b, l = x.shape
    n_emb, emb_dim = embedding_matrix.shape
    return _make_gather(b, l, emb_dim)(embedding_matrix, x.astype(jnp.int32))
